# baseline (speedup 1.0000x reference)
import functools
import jax
import jax.numpy as jnp
from jax import lax
from jax.experimental import pallas as pl
from jax.experimental.pallas import tpu as pltpu
from jax.experimental.pallas import tpu_sc as plsc

D_MODEL = 768
D_FF = 2048
E = 8
E_PAD = 16
N = 2048
T = 128
M_PAD = N * 2 + E * T
N_TILES = M_PAD // T
N_TILES_PAD = 64
BLK = 512
NW = 32
TPW = N // NW


def _k1_body(x_ref, rw_ref, eA_ref, eB_ref, rA_ref, rB_ref,
             pAs_ref, pBs_ref, off_ref, tmap_ref, run_ref):
    i = pl.program_id(0)

    @pl.when(i == 0)
    def _():
        run_ref[...] = jnp.zeros_like(run_ref)

    xb = x_ref[...]
    rw = rw_ref[...]
    logits8 = lax.dot_general(xb, rw, (((1,), (1,)), ((), ())),
                              preferred_element_type=jnp.float32)
    logits = jnp.concatenate(
        [logits8, jnp.full((BLK, E_PAD - E), -1e30, jnp.float32)], axis=1)
    lane = lax.broadcasted_iota(jnp.int32, (BLK, E_PAD), 1)
    valid = lane < E
    logits = jnp.clip(logits, -10000.0, 10000.0)
    lm = jnp.where(valid, logits, -1e30)
    m = jnp.max(lm, axis=-1, keepdims=True)
    ex = jnp.where(valid, jnp.exp(lm - m), 0.0)
    probs = ex / jnp.sum(ex, axis=-1, keepdims=True)
    probs = jnp.clip(probs, 1e-8, 1.0)
    probs = jnp.where(valid, probs, 0.0)

    lcol = lax.broadcasted_iota(jnp.int32, (E_PAD, 1), 0)
    pw = lax.shift_left(jnp.int32(1), 15 - lcol).astype(jnp.float32)

    def first_eq_lane(eq):
        bits = lax.dot_general(eq.astype(jnp.float32), pw,
                               (((1,), (0,)), ((), ())),
                               preferred_element_type=jnp.float32)
        e_exp = (lax.shift_right_logical(
            lax.bitcast_convert_type(bits, jnp.int32), 23) - 127)
        return 15 - e_exp

    m1 = jnp.max(probs, axis=-1, keepdims=True)
    i1 = first_eq_lane(probs == m1)[:, 0]
    ohA = lane == i1[:, None]
    masked = jnp.where(ohA | ~valid, -1.0, probs)
    m2 = jnp.max(masked, axis=-1, keepdims=True)
    i2 = first_eq_lane(masked == m2)[:, 0]
    ohB = lane == i2[:, None]

    p1 = m1[:, 0]
    p2 = m2[:, 0]
    s = p1 + p2
    pA = p1 / s
    pB = p2 / s

    ohAf = ohA.astype(jnp.float32)
    ohBf = ohB.astype(jnp.float32)
    Mf = ohAf + ohBf
    ri = lax.broadcasted_iota(jnp.int32, (BLK, BLK), 0)
    ci = lax.broadcasted_iota(jnp.int32, (BLK, BLK), 1)
    tril = (ri > ci).astype(jnp.float32)
    C = lax.dot_general(tril, Mf, (((1,), (0,)), ((), ())),
                        preferred_element_type=jnp.float32) + run_ref[...]
    rA = jnp.sum(C * ohAf, axis=-1).astype(jnp.int32)
    rB = jnp.sum(C * ohBf, axis=-1).astype(jnp.int32)
    run_ref[...] = run_ref[...] + jnp.sum(Mf, axis=0, keepdims=True)

    @pl.when(i == N // BLK - 1)
    def _():
        _k1_maps(run_ref, off_ref, tmap_ref)

    eA_ref[...] = i1[None, None, :]
    eB_ref[...] = i2[None, None, :]
    rA_ref[...] = rA[None, None, :]
    rB_ref[...] = rB[None, None, :]
    pAs_ref[...] = jnp.broadcast_to(pA[:, None], (BLK, E_PAD))
    pBs_ref[...] = jnp.broadcast_to(pB[:, None], (BLK, E_PAD))


def _k1_maps(run_ref, off_ref, tmap_ref):
    cnt_i = run_ref[...].astype(jnp.int32)
    padded = ((cnt_i + (T - 1)) & (-T)).astype(jnp.float32)
    ri16 = lax.broadcasted_iota(jnp.int32, (E_PAD, E_PAD), 0)
    ci16 = lax.broadcasted_iota(jnp.int32, (E_PAD, E_PAD), 1)
    pre = (ri16 < ci16).astype(jnp.float32)
    off = lax.dot_general(
        padded, pre, (((1,), (0,)), ((), ())),
        preferred_element_type=jnp.float32).astype(jnp.int32)
    off_ref[...] = off
    padded_i = (cnt_i + (T - 1)) & (-T)
    bt = (off + padded_i) // T
    ct = (off + cnt_i + (T - 1)) // T
    s_i = lax.broadcasted_iota(jnp.int32, (N_TILES_PAD, E_PAD), 0)
    l_e = lax.broadcasted_iota(jnp.int32, (N_TILES_PAD, E_PAD), 1)
    bt_b = jnp.broadcast_to(bt, (N_TILES_PAD, E_PAD))
    te = jnp.sum((s_i >= bt_b).astype(jnp.int32), axis=1)
    te = jnp.minimum(te, E - 1)
    ne_row = jnp.broadcast_to(cnt_i > 0, (N_TILES_PAD, E_PAD))
    is_tail = jnp.min(jnp.where(s_i >= bt_b, 1, 0), axis=1) > 0
    last_ne = jnp.max(jnp.where(ne_row,
                                lax.broadcasted_iota(
                                    jnp.int32, (N_TILES_PAD, E_PAD), 1),
                                0), axis=1)
    te = jnp.where(is_tail, last_ne, te)
    has_real = jnp.sum(((s_i >= jnp.broadcast_to(off // T, (N_TILES_PAD, E_PAD)))
                        & (s_i < jnp.broadcast_to(ct, (N_TILES_PAD, E_PAD)))
                        ).astype(jnp.int32), axis=1)
    nonempty = jnp.broadcast_to(cnt_i > 0, (N_TILES_PAD, E_PAD))
    gid = jnp.sum(((l_e < te[:, None]) & nonempty).astype(jnp.int32), axis=1)
    nxt1 = jnp.min(jnp.where((l_e > te[:, None]) & nonempty, l_e, E_PAD),
                   axis=1)
    nxt1 = jnp.where(nxt1 == E_PAD, te, nxt1)
    nxt2 = jnp.min(jnp.where((l_e > nxt1[:, None]) & nonempty, l_e, E_PAD),
                   axis=1)
    nxt2 = jnp.where(nxt2 == E_PAD, nxt1, nxt2)
    skip = jnp.where(has_real > 0, 0, 8)
    slot = gid - (gid // 3) * 3
    tmap_ref[...] = (te | skip | (slot << 4) | (nxt1 << 6)
                     | (nxt2 << 9))[None, :]


def _run_k1(x, router_w):
    nblk = N // BLK
    out_shapes = (
        jax.ShapeDtypeStruct((nblk, 1, BLK), jnp.int32),
        jax.ShapeDtypeStruct((nblk, 1, BLK), jnp.int32),
        jax.ShapeDtypeStruct((nblk, 1, BLK), jnp.int32),
        jax.ShapeDtypeStruct((nblk, 1, BLK), jnp.int32),
        jax.ShapeDtypeStruct((N, E_PAD), jnp.float32),
        jax.ShapeDtypeStruct((N, E_PAD), jnp.float32),
        jax.ShapeDtypeStruct((1, E_PAD), jnp.int32),
        jax.ShapeDtypeStruct((1, N_TILES_PAD), jnp.int32),
    )
    row_spec = pl.BlockSpec((1, 1, BLK), lambda i: (i, 0, 0))
    splat_spec = pl.BlockSpec((BLK, E_PAD), lambda i: (i, 0))
    return pl.pallas_call(
        _k1_body,
        grid=(nblk,),
        in_specs=[
            pl.BlockSpec((BLK, D_MODEL), lambda i: (i, 0)),
            pl.BlockSpec((E, D_MODEL), lambda i: (0, 0)),
        ],
        out_specs=(row_spec, row_spec, row_spec, row_spec,
                   splat_spec, splat_spec,
                   pl.BlockSpec((1, E_PAD), lambda i: (0, 0)),
                   pl.BlockSpec((1, N_TILES_PAD), lambda i: (0, 0))),
        out_shape=out_shapes,
        scratch_shapes=[pltpu.VMEM((1, E_PAD), jnp.float32)],
    )(x, router_w)


def _k2_body(x_hbm, eA_hbm, eB_hbm, rA_hbm, rB_hbm, off_hbm,
             xs_hbm, posA_hbm, posB_hbm,
             off_v, eA_v, eB_v, rA_v, rB_v, pA_v, pB_v, xrows_v,
             semX, semA, semB):
    wid = lax.axis_index("s") * 2 + lax.axis_index("c")
    base = wid * TPW

    ld = pltpu.async_copy(x_hbm.at[pl.ds(base, TPW)], xrows_v, semX)

    pltpu.sync_copy(off_hbm, off_v)
    pltpu.sync_copy(eA_hbm.at[pl.ds(base, TPW)], eA_v)
    pltpu.sync_copy(eB_hbm.at[pl.ds(base, TPW)], eB_v)
    pltpu.sync_copy(rA_hbm.at[pl.ds(base, TPW)], rA_v)
    pltpu.sync_copy(rB_hbm.at[pl.ds(base, TPW)], rB_v)

    off = off_v[...]
    for c in range(TPW // 16):
        sl = pl.ds(c * 16, 16)
        pA_v[sl] = off.at[eA_v[sl]].get(mode="promise_in_bounds") + rA_v[sl]
        pB_v[sl] = off.at[eB_v[sl]].get(mode="promise_in_bounds") + rB_v[sl]

    ld.wait()
    scA = pltpu.async_copy(xrows_v, xs_hbm.at[pA_v], semA)
    scB = pltpu.async_copy(xrows_v, xs_hbm.at[pB_v], semB)
    pltpu.sync_copy(pA_v, posA_hbm.at[pl.ds(base, TPW)])
    pltpu.sync_copy(pB_v, posB_hbm.at[pl.ds(base, TPW)])
    scA.wait()
    scB.wait()


def _run_k2(x, eA, eB, rA, rB, off):
    mesh = plsc.VectorSubcoreMesh(core_axis_name="c", subcore_axis_name="s")
    k = functools.partial(
        pl.kernel,
        mesh=mesh,
        out_type=[
            jax.ShapeDtypeStruct((M_PAD, D_MODEL), jnp.float32),
            jax.ShapeDtypeStruct((N,), jnp.int32),
            jax.ShapeDtypeStruct((N,), jnp.int32),
        ],
        scratch_types=[
            pltpu.VMEM((E_PAD,), jnp.int32),
            pltpu.VMEM((TPW,), jnp.int32),
            pltpu.VMEM((TPW,), jnp.int32),
            pltpu.VMEM((TPW,), jnp.int32),
            pltpu.VMEM((TPW,), jnp.int32),
            pltpu.VMEM((TPW,), jnp.int32),
            pltpu.VMEM((TPW,), jnp.int32),
            pltpu.VMEM((TPW, D_MODEL), jnp.float32),
            pltpu.SemaphoreType.DMA,
            pltpu.SemaphoreType.DMA,
            pltpu.SemaphoreType.DMA,
        ],
    )(_k2_body)
    return k(x, eA, eB, rA, rB, off)


def _k3_body(m_ref, xs_ref, w1_hbm, w2_hbm, ys_ref, w1b, w2b, sem1, sem2):
    i = pl.program_id(0)
    enc = m_ref[i]
    te = enc & 7
    skip = (enc >> 3) & 1
    slot = (enc >> 4) & 3
    nxt1 = (enc >> 6) & 7
    nxt2 = (enc >> 9) & 7
    prev_enc = m_ref[jnp.maximum(i - 1, 0)]
    first = jnp.logical_or(i == 0, (prev_enc & 7) != te)

    def fetch(e, s):
        pltpu.make_async_copy(w1_hbm.at[e], w1b.at[s], sem1.at[s]).start()
        pltpu.make_async_copy(w2_hbm.at[e], w2b.at[s], sem2.at[s]).start()

    @pl.when(i == 0)
    def _():
        fetch(te, slot)

        @pl.when(nxt1 != te)
        def _():
            fetch(nxt1, 1)

    s2 = slot + 2
    s2 = jnp.where(s2 >= 3, s2 - 3, s2)

    @pl.when(jnp.logical_and(first, nxt2 != nxt1))
    def _():
        fetch(nxt2, s2)

    @pl.when(first)
    def _():
        pltpu.make_async_copy(w1_hbm.at[te], w1b.at[slot],
                              sem1.at[slot]).wait()
        pltpu.make_async_copy(w2_hbm.at[te], w2b.at[slot],
                              sem2.at[slot]).wait()

    @pl.when(skip == 0)
    def _():
        xt = xs_ref[...]
        h = lax.dot_general(xt, w1b[slot], (((1,), (1,)), ((), ())),
                            preferred_element_type=jnp.float32)
        h = jax.nn.gelu(h)
        ys_ref[...] = lax.dot_general(h, w2b[slot], (((1,), (1,)), ((), ())),
                                      preferred_element_type=jnp.float32)


def _run_k3(tile_expert, xs, w1, w2):
    grid_spec = pltpu.PrefetchScalarGridSpec(
        num_scalar_prefetch=1,
        grid=(N_TILES,),
        in_specs=[
            pl.BlockSpec((T, D_MODEL), lambda i, m: (i, 0)),
            pl.BlockSpec(memory_space=pl.ANY),
            pl.BlockSpec(memory_space=pl.ANY),
        ],
        out_specs=pl.BlockSpec((T, D_MODEL), lambda i, m: (i, 0)),
        scratch_shapes=[
            pltpu.VMEM((3, D_FF, D_MODEL), jnp.float32),
            pltpu.VMEM((3, D_MODEL, D_FF), jnp.float32),
            pltpu.SemaphoreType.DMA((3,)),
            pltpu.SemaphoreType.DMA((3,)),
        ],
    )
    return pl.pallas_call(
        _k3_body,
        grid_spec=grid_spec,
        out_shape=jax.ShapeDtypeStruct((M_PAD, D_MODEL), jnp.float32),
    )(tile_expert, xs, w1, w2)


HALF = TPW // 2


def _k4_body(ys_hbm, posA_hbm, posB_hbm, pAs_hbm, pBs_hbm, out_hbm,
             idxA_v, idxB_v, yA_v, yB_v, pA_v, pB_v,
             semA0, semB0, semA1, semB1, semO):
    wid = lax.axis_index("s") * 2 + lax.axis_index("c")
    base = wid * TPW

    pltpu.sync_copy(posA_hbm.at[pl.ds(base, TPW)], idxA_v)
    pltpu.sync_copy(posB_hbm.at[pl.ds(base, TPW)], idxB_v)
    h0, h1 = pl.ds(0, HALF), pl.ds(HALF, HALF)
    cA0 = pltpu.async_copy(ys_hbm.at[idxA_v.at[h0]], yA_v.at[h0], semA0)
    cB0 = pltpu.async_copy(ys_hbm.at[idxB_v.at[h0]], yB_v.at[h0], semB0)
    cA1 = pltpu.async_copy(ys_hbm.at[idxA_v.at[h1]], yA_v.at[h1], semA1)
    cB1 = pltpu.async_copy(ys_hbm.at[idxB_v.at[h1]], yB_v.at[h1], semB1)
    pltpu.sync_copy(pAs_hbm.at[pl.ds(base, TPW)], pA_v)
    pltpu.sync_copy(pBs_hbm.at[pl.ds(base, TPW)], pB_v)

    def body(j, _):
        pa = pA_v[j, :]
        pb = pB_v[j, :]
        for v in range(D_MODEL // 16):
            sl = pl.ds(v * 16, 16)
            yA_v[j, sl] = pa * yA_v[j, sl] + pb * yB_v[j, sl]
        return 0

    cA0.wait()
    cB0.wait()
    lax.fori_loop(0, HALF, body, 0)
    st0 = pltpu.async_copy(yA_v.at[h0], out_hbm.at[pl.ds(base, HALF)], semO)
    cA1.wait()
    cB1.wait()
    lax.fori_loop(HALF, TPW, body, 0)
    st0.wait()
    pltpu.sync_copy(yA_v.at[h1], out_hbm.at[pl.ds(base + HALF, HALF)])


def _run_k4(ys, posA, posB, pAs, pBs):
    mesh = plsc.VectorSubcoreMesh(core_axis_name="c", subcore_axis_name="s")
    k = functools.partial(
        pl.kernel,
        mesh=mesh,
        out_type=[jax.ShapeDtypeStruct((N, D_MODEL), jnp.float32)],
        scratch_types=[
            pltpu.VMEM((TPW,), jnp.int32),
            pltpu.VMEM((TPW,), jnp.int32),
            pltpu.VMEM((TPW, D_MODEL), jnp.float32),
            pltpu.VMEM((TPW, D_MODEL), jnp.float32),
            pltpu.VMEM((TPW, E_PAD), jnp.float32),
            pltpu.VMEM((TPW, E_PAD), jnp.float32),
            pltpu.SemaphoreType.DMA,
            pltpu.SemaphoreType.DMA,
            pltpu.SemaphoreType.DMA,
            pltpu.SemaphoreType.DMA,
            pltpu.SemaphoreType.DMA,
        ],
    )(_k4_body)
    (out,) = k(ys, posA, posB, pAs, pBs)
    return out


@jax.jit
def kernel(x, router_w, w1, w2):
    eA, eB, rA, rB, pAs, pBs, off, tmap = _run_k1(x, router_w)
    xs, posA, posB = _run_k2(x, eA.reshape(N), eB.reshape(N),
                             rA.reshape(N), rB.reshape(N),
                             off.reshape(E_PAD))
    ys = _run_k3(tmap.reshape(N_TILES_PAD), xs, w1, w2)
    return _run_k4(ys, posA, posB, pAs, pBs)
